# initial kernel scaffold (unmeasured)
import jax
import jax.numpy as jnp
from jax import lax
from jax.experimental import pallas as pl
from jax.experimental.pallas import tpu as pltpu

N_DEV = 4


def _gemm_body(x_ref, w_ref, y_ref, amax_ref):
    j = pl.program_id(0)
    y = jnp.dot(
        x_ref[...].astype(jnp.bfloat16),
        w_ref[...].astype(jnp.bfloat16),
        preferred_element_type=jnp.float32,
    )
    y_ref[...] = y
    m = jnp.max(jnp.abs(y))

    @pl.when(j == 0)
    def _():
        amax_ref[...] = jnp.zeros_like(amax_ref)

    amax_ref[...] = jnp.maximum(amax_ref[...], m)


def _local_gemm(x, w):
    m_per, k = x.shape
    _, n = w.shape
    nb = 8
    bn = n // nb
    return pl.pallas_call(
        _gemm_body,
        grid=(nb,),
        in_specs=[
            pl.BlockSpec((m_per, k), lambda j: (0, 0)),
            pl.BlockSpec((k, bn), lambda j: (0, j)),
        ],
        out_specs=[
            pl.BlockSpec((m_per, bn), lambda j: (0, j)),
            pl.BlockSpec((8, 128), lambda j: (0, 0)),
        ],
        out_shape=[
            jax.ShapeDtypeStruct((m_per, n), jnp.float32),
            jax.ShapeDtypeStruct((8, 128), jnp.float32),
        ],
    )(x, w)


def _a2a_body(
    y_ref,
    amax_ref,
    out_ref,
    gmax_ref,
    amax_all,
    send_sems,
    recv_sems,
    a_send_sems,
    a_recv_sems,
    copy_sem,
):
    my = lax.axis_index("i")
    m_per = y_ref.shape[0]
    n_per = out_ref.shape[1]

    barrier_sem = pltpu.get_barrier_semaphore()
    for d in range(1, N_DEV):
        peer = lax.rem(my + d, N_DEV)
        pl.semaphore_signal(
            barrier_sem, inc=1,
            device_id=(peer,), device_id_type=pl.DeviceIdType.MESH,
        )
    pl.semaphore_wait(barrier_sem, N_DEV - 1)

    rdmas = []
    for d in range(1, N_DEV):
        peer = lax.rem(my + d, N_DEV)
        rdma = pltpu.make_async_remote_copy(
            src_ref=y_ref.at[:, pl.ds(peer * n_per, n_per)],
            dst_ref=out_ref.at[pl.ds(my * m_per, m_per), :],
            send_sem=send_sems.at[d],
            recv_sem=recv_sems.at[d],
            device_id=(peer,),
            device_id_type=pl.DeviceIdType.MESH,
        )
        rdma.start()
        rdmas.append(rdma)

    a_rdmas = []
    for d in range(1, N_DEV):
        peer = lax.rem(my + d, N_DEV)
        ar = pltpu.make_async_remote_copy(
            src_ref=amax_ref,
            dst_ref=amax_all.at[d],
            send_sem=a_send_sems.at[d],
            recv_sem=a_recv_sems.at[d],
            device_id=(peer,),
            device_id_type=pl.DeviceIdType.MESH,
        )
        ar.start()
        a_rdmas.append(ar)

    cp = pltpu.make_async_copy(
        y_ref.at[:, pl.ds(my * n_per, n_per)],
        out_ref.at[pl.ds(my * m_per, m_per), :],
        copy_sem,
    )
    cp.start()
    cp.wait()

    for r in rdmas:
        r.wait()
    for r in a_rdmas:
        r.wait()

    g = amax_ref[0, 0]
    for d in range(1, N_DEV):
        g = jnp.maximum(g, amax_all[d, 0, 0])
    gmax_ref[...] = jnp.full_like(gmax_ref, g)


def _a2a(y, amax):
    m_per, n = y.shape
    n_per = n // N_DEV
    return pl.pallas_call(
        _a2a_body,
        in_specs=[
            pl.BlockSpec(memory_space=pltpu.ANY),
            pl.BlockSpec(memory_space=pltpu.VMEM),
        ],
        out_specs=[
            pl.BlockSpec(memory_space=pltpu.ANY),
            pl.BlockSpec(memory_space=pltpu.VMEM),
        ],
        out_shape=[
            jax.ShapeDtypeStruct((N_DEV * m_per, n_per), jnp.float32),
            jax.ShapeDtypeStruct((8, 128), jnp.float32),
        ],
        scratch_shapes=[
            pltpu.VMEM((N_DEV, 8, 128), jnp.float32),
            pltpu.SemaphoreType.DMA((N_DEV,)),
            pltpu.SemaphoreType.DMA((N_DEV,)),
            pltpu.SemaphoreType.DMA((N_DEV,)),
            pltpu.SemaphoreType.DMA((N_DEV,)),
            pltpu.SemaphoreType.DMA,
        ],
        compiler_params=pltpu.CompilerParams(collective_id=0),
    )(y, amax)


def _quant_body(y_ref, gmax_ref, out_ref):
    g = gmax_ref[0, 0]
    scale = g / 127.0
    q = jnp.clip(jnp.round(y_ref[...] * (127.0 / g)), -127.0, 127.0)
    out_ref[...] = q * scale


def _quant(y_ex, gmax):
    m, n_per = y_ex.shape
    nb = 4
    bm = m // nb
    return pl.pallas_call(
        _quant_body,
        grid=(nb,),
        in_specs=[
            pl.BlockSpec((bm, n_per), lambda j: (j, 0)),
            pl.BlockSpec((8, 128), lambda j: (0, 0)),
        ],
        out_specs=pl.BlockSpec((bm, n_per), lambda j: (j, 0)),
        out_shape=jax.ShapeDtypeStruct((m, n_per), jnp.float32),
    )(y_ex, gmax)


def kernel(x, w_mat):
    y_local, amax_local = _local_gemm(x, w_mat)
    y_ex, gmax = _a2a(y_local, amax_local)
    return _quant(y_ex, gmax)


# baseline (device time: 372342 ns/iter reference)
import jax
import jax.numpy as jnp
from jax import lax
from jax.experimental import pallas as pl
from jax.experimental.pallas import tpu as pltpu

N_DEV = 4


def _gemm_body(x_ref, w_ref, y_ref, amax_ref):
    j = pl.program_id(0)
    y = jnp.dot(
        x_ref[...].astype(jnp.bfloat16),
        w_ref[...].astype(jnp.bfloat16),
        preferred_element_type=jnp.float32,
    )
    y_ref[...] = y
    m = jnp.max(jnp.abs(y))

    @pl.when(j == 0)
    def _():
        amax_ref[...] = jnp.zeros_like(amax_ref)

    amax_ref[...] = jnp.maximum(amax_ref[...], m)


def _local_gemm(x, w):
    m_per, k = x.shape
    _, n = w.shape
    nb = 16
    bn = n // nb
    return pl.pallas_call(
        _gemm_body,
        grid=(nb,),
        in_specs=[
            pl.BlockSpec((m_per, k), lambda j: (0, 0)),
            pl.BlockSpec((k, bn), lambda j: (0, j)),
        ],
        out_specs=[
            pl.BlockSpec((m_per, bn), lambda j: (0, j)),
            pl.BlockSpec((8, 128), lambda j: (0, 0)),
        ],
        out_shape=[
            jax.ShapeDtypeStruct((m_per, n), jnp.float32),
            jax.ShapeDtypeStruct((8, 128), jnp.float32),
        ],
        compiler_params=pltpu.CompilerParams(
            vmem_limit_bytes=96 * 1024 * 1024,
        ),
    )(x, w)


def _a2a_body(
    y_ref,
    amax_ref,
    out_ref,
    gmax_ref,
    amax_all,
    send_sems,
    recv_sems,
    a_send_sems,
    a_recv_sems,
    copy_sem,
):
    my = lax.axis_index("i")
    m_per = y_ref.shape[0]
    n_per = out_ref.shape[1]

    barrier_sem = pltpu.get_barrier_semaphore()
    for d in range(1, N_DEV):
        peer = lax.rem(my + d, N_DEV)
        pl.semaphore_signal(
            barrier_sem, inc=1,
            device_id=(peer,), device_id_type=pl.DeviceIdType.MESH,
        )
    pl.semaphore_wait(barrier_sem, N_DEV - 1)

    rdmas = []
    for d in range(1, N_DEV):
        peer = lax.rem(my + d, N_DEV)
        rdma = pltpu.make_async_remote_copy(
            src_ref=y_ref.at[:, pl.ds(peer * n_per, n_per)],
            dst_ref=out_ref.at[pl.ds(my * m_per, m_per), :],
            send_sem=send_sems.at[d],
            recv_sem=recv_sems.at[d],
            device_id=(peer,),
            device_id_type=pl.DeviceIdType.MESH,
        )
        rdma.start()
        rdmas.append(rdma)

    a_rdmas = []
    for d in range(1, N_DEV):
        peer = lax.rem(my + d, N_DEV)
        ar = pltpu.make_async_remote_copy(
            src_ref=amax_ref,
            dst_ref=amax_all.at[d],
            send_sem=a_send_sems.at[d],
            recv_sem=a_recv_sems.at[d],
            device_id=(peer,),
            device_id_type=pl.DeviceIdType.MESH,
        )
        ar.start()
        a_rdmas.append(ar)

    cp = pltpu.make_async_copy(
        y_ref.at[:, pl.ds(my * n_per, n_per)],
        out_ref.at[pl.ds(my * m_per, m_per), :],
        copy_sem,
    )
    cp.start()
    cp.wait()

    for r in rdmas:
        r.wait()
    for r in a_rdmas:
        r.wait()

    g = amax_ref[0, 0]
    for d in range(1, N_DEV):
        g = jnp.maximum(g, amax_all[d, 0, 0])
    gmax_ref[...] = jnp.full_like(gmax_ref, g)


def _a2a(y, amax):
    m_per, n = y.shape
    n_per = n // N_DEV
    return pl.pallas_call(
        _a2a_body,
        in_specs=[
            pl.BlockSpec(memory_space=pl.ANY),
            pl.BlockSpec(memory_space=pltpu.VMEM),
        ],
        out_specs=[
            pl.BlockSpec(memory_space=pl.ANY),
            pl.BlockSpec(memory_space=pltpu.VMEM),
        ],
        out_shape=[
            jax.ShapeDtypeStruct((N_DEV * m_per, n_per), jnp.float32),
            jax.ShapeDtypeStruct((8, 128), jnp.float32),
        ],
        scratch_shapes=[
            pltpu.VMEM((N_DEV, 8, 128), jnp.float32),
            pltpu.SemaphoreType.DMA((N_DEV,)),
            pltpu.SemaphoreType.DMA((N_DEV,)),
            pltpu.SemaphoreType.DMA((N_DEV,)),
            pltpu.SemaphoreType.DMA((N_DEV,)),
            pltpu.SemaphoreType.DMA,
        ],
        compiler_params=pltpu.CompilerParams(collective_id=0),
    )(y, amax)


def _quant_body(y_ref, gmax_ref, out_ref):
    g = gmax_ref[0, 0]
    scale = g / 127.0
    q = jnp.clip(jnp.round(y_ref[...] * (127.0 / g)), -127.0, 127.0)
    out_ref[...] = q * scale


def _quant(y_ex, gmax):
    m, n_per = y_ex.shape
    nb = 8
    bm = m // nb
    return pl.pallas_call(
        _quant_body,
        grid=(nb,),
        in_specs=[
            pl.BlockSpec((bm, n_per), lambda j: (j, 0)),
            pl.BlockSpec((8, 128), lambda j: (0, 0)),
        ],
        out_specs=pl.BlockSpec((bm, n_per), lambda j: (j, 0)),
        out_shape=jax.ShapeDtypeStruct((m, n_per), jnp.float32),
    )(y_ex, gmax)


def kernel(x, w_mat):
    y_local, amax_local = _local_gemm(x, w_mat)
    y_ex, gmax = _a2a(y_local, amax_local)
    return _quant(y_ex, gmax)


# device time: 210397 ns/iter; 1.7697x vs baseline; 1.7697x over previous
import jax
import jax.numpy as jnp
from jax import lax
from jax.experimental import pallas as pl
from jax.experimental.pallas import tpu as pltpu

N_DEV = 4


def _gemm_body(x_ref, w_ref, y_ref, amax_ref):
    j = pl.program_id(0)
    y = jnp.dot(
        x_ref[...].astype(jnp.bfloat16),
        w_ref[...].astype(jnp.bfloat16),
        preferred_element_type=jnp.float32,
    )
    y_ref[...] = y.astype(jnp.bfloat16)
    m = jnp.max(jnp.abs(y))

    @pl.when(j == 0)
    def _():
        amax_ref[...] = jnp.zeros_like(amax_ref)

    amax_ref[...] = jnp.maximum(amax_ref[...], m)


def _local_gemm(x, w):
    m_per, k = x.shape
    _, n = w.shape
    nb = 16
    bn = n // nb
    return pl.pallas_call(
        _gemm_body,
        grid=(nb,),
        in_specs=[
            pl.BlockSpec((m_per, k), lambda j: (0, 0)),
            pl.BlockSpec((k, bn), lambda j: (0, j)),
        ],
        out_specs=[
            pl.BlockSpec((m_per, bn), lambda j: (0, j)),
            pl.BlockSpec((8, 128), lambda j: (0, 0)),
        ],
        out_shape=[
            jax.ShapeDtypeStruct((m_per, n), jnp.bfloat16),
            jax.ShapeDtypeStruct((8, 128), jnp.float32),
        ],
        compiler_params=pltpu.CompilerParams(
            vmem_limit_bytes=60 * 1024 * 1024,
        ),
    )(x, w)


def _a2a_body(
    y_ref,
    amax_ref,
    out_ref,
    gmax_ref,
    amax_all,
    send_sems,
    recv_sems,
    a_send_sems,
    a_recv_sems,
    copy_sem,
):
    my = lax.axis_index("i")
    m_per = y_ref.shape[0]
    n_per = out_ref.shape[1]

    barrier_sem = pltpu.get_barrier_semaphore()
    for d in range(1, N_DEV):
        peer = lax.rem(my + d, N_DEV)
        pl.semaphore_signal(
            barrier_sem, inc=1,
            device_id=(peer,), device_id_type=pl.DeviceIdType.MESH,
        )
    pl.semaphore_wait(barrier_sem, N_DEV - 1)

    rdmas = []
    for d in range(1, N_DEV):
        peer = lax.rem(my + d, N_DEV)
        rdma = pltpu.make_async_remote_copy(
            src_ref=y_ref.at[:, pl.ds(peer * n_per, n_per)],
            dst_ref=out_ref.at[pl.ds(my * m_per, m_per), :],
            send_sem=send_sems.at[d],
            recv_sem=recv_sems.at[d],
            device_id=(peer,),
            device_id_type=pl.DeviceIdType.MESH,
        )
        rdma.start()
        rdmas.append(rdma)

    a_rdmas = []
    for d in range(1, N_DEV):
        peer = lax.rem(my + d, N_DEV)
        ar = pltpu.make_async_remote_copy(
            src_ref=amax_ref,
            dst_ref=amax_all.at[d],
            send_sem=a_send_sems.at[d],
            recv_sem=a_recv_sems.at[d],
            device_id=(peer,),
            device_id_type=pl.DeviceIdType.MESH,
        )
        ar.start()
        a_rdmas.append(ar)

    cp = pltpu.make_async_copy(
        y_ref.at[:, pl.ds(my * n_per, n_per)],
        out_ref.at[pl.ds(my * m_per, m_per), :],
        copy_sem,
    )
    cp.start()
    cp.wait()

    for r in rdmas:
        r.wait()
    for r in a_rdmas:
        r.wait()

    g = amax_ref[0, 0]
    for d in range(1, N_DEV):
        g = jnp.maximum(g, amax_all[d, 0, 0])
    gmax_ref[...] = jnp.full_like(gmax_ref, g)


def _a2a(y, amax):
    m_per, n = y.shape
    n_per = n // N_DEV
    return pl.pallas_call(
        _a2a_body,
        in_specs=[
            pl.BlockSpec(memory_space=pl.ANY),
            pl.BlockSpec(memory_space=pltpu.VMEM),
        ],
        out_specs=[
            pl.BlockSpec(memory_space=pl.ANY),
            pl.BlockSpec(memory_space=pltpu.VMEM),
        ],
        out_shape=[
            jax.ShapeDtypeStruct((N_DEV * m_per, n_per), y.dtype),
            jax.ShapeDtypeStruct((8, 128), jnp.float32),
        ],
        scratch_shapes=[
            pltpu.VMEM((N_DEV, 8, 128), jnp.float32),
            pltpu.SemaphoreType.DMA((N_DEV,)),
            pltpu.SemaphoreType.DMA((N_DEV,)),
            pltpu.SemaphoreType.DMA((N_DEV,)),
            pltpu.SemaphoreType.DMA((N_DEV,)),
            pltpu.SemaphoreType.DMA,
        ],
        compiler_params=pltpu.CompilerParams(collective_id=0),
    )(y, amax)


def _quant_body(y_ref, gmax_ref, out_ref):
    g = gmax_ref[0, 0]
    scale = g / 127.0
    y = y_ref[...].astype(jnp.float32)
    q = jnp.clip(jnp.round(y * (127.0 / g)), -127.0, 127.0)
    out_ref[...] = q * scale


def _quant(y_ex, gmax):
    m, n_per = y_ex.shape
    nb = 8
    bm = m // nb
    return pl.pallas_call(
        _quant_body,
        grid=(nb,),
        in_specs=[
            pl.BlockSpec((bm, n_per), lambda j: (j, 0)),
            pl.BlockSpec((8, 128), lambda j: (0, 0)),
        ],
        out_specs=pl.BlockSpec((bm, n_per), lambda j: (j, 0)),
        out_shape=jax.ShapeDtypeStruct((m, n_per), jnp.float32),
    )(y_ex, gmax)


def kernel(x, w_mat):
    y_local, amax_local = _local_gemm(x, w_mat)
    y_ex, gmax = _a2a(y_local, amax_local)
    return _quant(y_ex, gmax)


# device time: 167064 ns/iter; 2.2287x vs baseline; 1.2594x over previous
import jax
import jax.numpy as jnp
from jax import lax
from jax.experimental import pallas as pl
from jax.experimental.pallas import tpu as pltpu

N_DEV = 4
KI = 4


def _fused_body(
    perm_ref,
    x_ref,
    w_ref,
    yex_ref,
    gmax_ref,
    xb_ref,
    ybuf_ref,
    amax_smem,
    amax_tile,
    amax_all,
    dsend, drecv,
    asend, arecv,
    copy_sem,
):
    j = pl.program_id(0)
    ki = pl.program_id(1)
    me = lax.axis_index("i")
    m_per = x_ref.shape[0]
    n_per = ybuf_ref.shape[2]
    bn = w_ref.shape[1]
    slot = lax.rem(j, 2)

    def _data_rdma(step, slot_):
        return pltpu.make_async_remote_copy(
            src_ref=ybuf_ref.at[slot_],
            dst_ref=yex_ref.at[pl.ds(me * m_per, m_per), :],
            send_sem=dsend.at[step],
            recv_sem=drecv.at[step],
            device_id=(perm_ref[step],),
            device_id_type=pl.DeviceIdType.MESH,
        )

    @pl.when(jnp.logical_and(j == 0, ki == 0))
    def _():
        barrier_sem = pltpu.get_barrier_semaphore()
        for d in range(1, N_DEV):
            peer = lax.rem(me + d, N_DEV)
            pl.semaphore_signal(
                barrier_sem, inc=1,
                device_id=(peer,), device_id_type=pl.DeviceIdType.MESH,
            )
        pl.semaphore_wait(barrier_sem, N_DEV - 1)
        xb_ref[...] = x_ref[...].astype(jnp.bfloat16)
        amax_smem[0] = 0.0

    @pl.when(jnp.logical_and(j >= 2, ki == 0))
    def _():
        _data_rdma(j - 2, slot).wait_send()

    y = jnp.dot(
        xb_ref[...],
        w_ref[...].astype(jnp.bfloat16),
        preferred_element_type=jnp.float32,
    )
    amax_smem[0] = jnp.maximum(amax_smem[0], jnp.max(jnp.abs(y)))
    ybuf_ref[slot, :, pl.ds(ki * bn, bn)] = y.astype(jnp.bfloat16)

    @pl.when(jnp.logical_and(ki == KI - 1, j < N_DEV - 1))
    def _():
        _data_rdma(j, slot).start()

    @pl.when(jnp.logical_and(ki == KI - 1, j == N_DEV - 1))
    def _():
        cp = pltpu.make_async_copy(
            ybuf_ref.at[slot],
            yex_ref.at[pl.ds(me * m_per, m_per), :],
            copy_sem,
        )
        cp.start()

        amax_tile[...] = jnp.full_like(amax_tile, amax_smem[0])
        a_rdmas = []
        for d in range(1, N_DEV):
            peer = lax.rem(me + d, N_DEV)
            ar = pltpu.make_async_remote_copy(
                src_ref=amax_tile,
                dst_ref=amax_all.at[d],
                send_sem=asend.at[d],
                recv_sem=arecv.at[d],
                device_id=(peer,),
                device_id_type=pl.DeviceIdType.MESH,
            )
            ar.start()
            a_rdmas.append(ar)

        cp.wait()
        _data_rdma(2, 0).wait_send()
        for step in range(N_DEV - 1):
            _data_rdma(step, 0).wait_recv()
        for ar in a_rdmas:
            ar.wait()

        g = amax_smem[0]
        for d in range(1, N_DEV):
            g = jnp.maximum(g, amax_all[d, 0, 0])
        gmax_ref[...] = jnp.full_like(gmax_ref, g)


def _fused_gemm_a2a(perm, x, w):
    m_per, k = x.shape
    _, n = w.shape
    n_per = n // N_DEV
    bn = n_per // KI
    grid_spec = pltpu.PrefetchScalarGridSpec(
        num_scalar_prefetch=1,
        grid=(N_DEV, KI),
        in_specs=[
            pl.BlockSpec((m_per, k), lambda j, ki, perm: (0, 0)),
            pl.BlockSpec((k, bn), lambda j, ki, perm: (0, perm[j] * KI + ki)),
        ],
        out_specs=[
            pl.BlockSpec(memory_space=pl.ANY),
            pl.BlockSpec((8, 128), lambda j, ki, perm: (0, 0)),
        ],
        scratch_shapes=[
            pltpu.VMEM((m_per, k), jnp.bfloat16),
            pltpu.VMEM((2, m_per, n_per), jnp.bfloat16),
            pltpu.SMEM((1,), jnp.float32),
            pltpu.VMEM((8, 128), jnp.float32),
            pltpu.VMEM((N_DEV, 8, 128), jnp.float32),
            pltpu.SemaphoreType.DMA((N_DEV - 1,)),
            pltpu.SemaphoreType.DMA((N_DEV - 1,)),
            pltpu.SemaphoreType.DMA((N_DEV,)),
            pltpu.SemaphoreType.DMA((N_DEV,)),
            pltpu.SemaphoreType.DMA,
        ],
    )
    return pl.pallas_call(
        _fused_body,
        grid_spec=grid_spec,
        out_shape=[
            jax.ShapeDtypeStruct((N_DEV * m_per, n_per), jnp.bfloat16),
            jax.ShapeDtypeStruct((8, 128), jnp.float32),
        ],
        compiler_params=pltpu.CompilerParams(
            collective_id=0,
            vmem_limit_bytes=60 * 1024 * 1024,
        ),
    )(perm, x, w)


def _quant_body(y_ref, gmax_ref, out_ref):
    g = gmax_ref[0, 0]
    scale = g / 127.0
    y = y_ref[...].astype(jnp.float32)
    q = jnp.clip(jnp.round(y * (127.0 / g)), -127.0, 127.0)
    out_ref[...] = q * scale


def _quant(y_ex, gmax):
    m, n_per = y_ex.shape
    nb = 8
    bm = m // nb
    return pl.pallas_call(
        _quant_body,
        grid=(nb,),
        in_specs=[
            pl.BlockSpec((bm, n_per), lambda j: (j, 0)),
            pl.BlockSpec((8, 128), lambda j: (0, 0)),
        ],
        out_specs=pl.BlockSpec((bm, n_per), lambda j: (j, 0)),
        out_shape=jax.ShapeDtypeStruct((m, n_per), jnp.float32),
    )(y_ex, gmax)


def kernel(x, w_mat):
    me = lax.axis_index("i")
    perm = lax.rem(me + 1 + jnp.arange(N_DEV, dtype=jnp.int32), N_DEV)
    y_ex, gmax = _fused_gemm_a2a(perm, x, w_mat)
    return _quant(y_ex, gmax)


# device time: 146684 ns/iter; 2.5384x vs baseline; 1.1389x over previous
import jax
import jax.numpy as jnp
from jax import lax
from jax.experimental import pallas as pl
from jax.experimental.pallas import tpu as pltpu

N_DEV = 4
KI = 4


def _fused_body(
    perm_ref,
    x_ref,
    w_ref,
    yex_ref,
    gmax_ref,
    ybuf_ref,
    amax_smem,
    amax_tile,
    amax_all,
    dsend, drecv,
    asend, arecv,
    copy_sem,
):
    j = pl.program_id(0)
    ki = pl.program_id(1)
    me = lax.axis_index("i")
    m_per = x_ref.shape[0]
    bn = w_ref.shape[1]

    def _data_rdma(step, slot_):
        return pltpu.make_async_remote_copy(
            src_ref=ybuf_ref.at[slot_],
            dst_ref=yex_ref.at[pl.ds(me * m_per, m_per), :],
            send_sem=dsend.at[step],
            recv_sem=drecv.at[step],
            device_id=(perm_ref[step],),
            device_id_type=pl.DeviceIdType.MESH,
        )

    @pl.when(jnp.logical_and(j == 0, ki == 0))
    def _():
        barrier_sem = pltpu.get_barrier_semaphore()
        for d in range(1, N_DEV):
            peer = lax.rem(me + d, N_DEV)
            pl.semaphore_signal(
                barrier_sem, inc=1,
                device_id=(peer,), device_id_type=pl.DeviceIdType.MESH,
            )
        pl.semaphore_wait(barrier_sem, N_DEV - 1)
        amax_smem[0] = 0.0

    y = jnp.dot(
        x_ref[...].astype(jnp.bfloat16),
        w_ref[...].astype(jnp.bfloat16),
        preferred_element_type=jnp.float32,
    )
    amax_smem[0] = jnp.maximum(amax_smem[0], jnp.max(jnp.abs(y)))
    ybuf_ref[j, :, pl.ds(ki * bn, bn)] = y.astype(jnp.bfloat16)

    @pl.when(jnp.logical_and(ki == KI - 1, j < N_DEV - 1))
    def _():
        _data_rdma(j, j).start()

    @pl.when(jnp.logical_and(ki == KI - 1, j == N_DEV - 1))
    def _():
        cp = pltpu.make_async_copy(
            ybuf_ref.at[j],
            yex_ref.at[pl.ds(me * m_per, m_per), :],
            copy_sem,
        )
        cp.start()

        amax_tile[...] = jnp.full_like(amax_tile, amax_smem[0])
        a_rdmas = []
        for d in range(1, N_DEV):
            peer = lax.rem(me + d, N_DEV)
            ar = pltpu.make_async_remote_copy(
                src_ref=amax_tile,
                dst_ref=amax_all.at[d],
                send_sem=asend.at[d],
                recv_sem=arecv.at[d],
                device_id=(peer,),
                device_id_type=pl.DeviceIdType.MESH,
            )
            ar.start()
            a_rdmas.append(ar)

        cp.wait()
        for step in range(N_DEV - 1):
            _data_rdma(step, step).wait_send()
        for step in range(N_DEV - 1):
            _data_rdma(step, step).wait_recv()
        for ar in a_rdmas:
            ar.wait()

        g = amax_smem[0]
        for d in range(1, N_DEV):
            g = jnp.maximum(g, amax_all[d, 0, 0])
        gmax_ref[...] = jnp.full_like(gmax_ref, g)


def _fused_gemm_a2a(perm, x, w):
    m_per, k = x.shape
    _, n = w.shape
    n_per = n // N_DEV
    bn = n_per // KI
    grid_spec = pltpu.PrefetchScalarGridSpec(
        num_scalar_prefetch=1,
        grid=(N_DEV, KI),
        in_specs=[
            pl.BlockSpec((m_per, k), lambda j, ki, perm: (0, 0)),
            pl.BlockSpec((k, bn), lambda j, ki, perm: (0, perm[j] * KI + ki)),
        ],
        out_specs=[
            pl.BlockSpec(memory_space=pl.ANY),
            pl.BlockSpec((8, 128), lambda j, ki, perm: (0, 0)),
        ],
        scratch_shapes=[
            pltpu.VMEM((N_DEV, m_per, n_per), jnp.bfloat16),
            pltpu.SMEM((1,), jnp.float32),
            pltpu.VMEM((8, 128), jnp.float32),
            pltpu.VMEM((N_DEV, 8, 128), jnp.float32),
            pltpu.SemaphoreType.DMA((N_DEV - 1,)),
            pltpu.SemaphoreType.DMA((N_DEV - 1,)),
            pltpu.SemaphoreType.DMA((N_DEV,)),
            pltpu.SemaphoreType.DMA((N_DEV,)),
            pltpu.SemaphoreType.DMA,
        ],
    )
    return pl.pallas_call(
        _fused_body,
        grid_spec=grid_spec,
        out_shape=[
            jax.ShapeDtypeStruct((N_DEV * m_per, n_per), jnp.bfloat16),
            jax.ShapeDtypeStruct((8, 128), jnp.float32),
        ],
        compiler_params=pltpu.CompilerParams(
            collective_id=0,
            vmem_limit_bytes=60 * 1024 * 1024,
        ),
    )(perm, x, w)


def _quant_body(y_ref, gmax_ref, out_ref):
    g = gmax_ref[0, 0]
    scale = g / 127.0
    y = y_ref[...].astype(jnp.float32)
    q = jnp.clip(jnp.round(y * (127.0 / g)), -127.0, 127.0)
    out_ref[...] = q * scale


def _quant(y_ex, gmax):
    m, n_per = y_ex.shape
    nb = 8
    bm = m // nb
    return pl.pallas_call(
        _quant_body,
        grid=(nb,),
        in_specs=[
            pl.BlockSpec((bm, n_per), lambda j: (j, 0)),
            pl.BlockSpec((8, 128), lambda j: (0, 0)),
        ],
        out_specs=pl.BlockSpec((bm, n_per), lambda j: (j, 0)),
        out_shape=jax.ShapeDtypeStruct((m, n_per), jnp.float32),
    )(y_ex, gmax)


def kernel(x, w_mat):
    me = lax.axis_index("i")
    perm = lax.rem(me + jnp.array([2, 1, 3, 0], dtype=jnp.int32), N_DEV)
    y_ex, gmax = _fused_gemm_a2a(perm, x, w_mat)
    return _quant(y_ex, gmax)


# device time: 136794 ns/iter; 2.7219x vs baseline; 1.0723x over previous
import jax
import jax.numpy as jnp
from jax import lax
from jax.experimental import pallas as pl
from jax.experimental.pallas import tpu as pltpu

N_DEV = 4
KI = 4


def _fused_body(
    perm_ref,
    x_ref,
    w_ref,
    yex_ref,
    gmax_ref,
    ybuf_ref,
    amax_smem,
    amax_tile,
    amax_all,
    dsend, drecv,
    asend, arecv,
    copy_sem,
):
    j = pl.program_id(0)
    ki = pl.program_id(1)
    me = lax.axis_index("i")
    m_per = x_ref.shape[0]
    bn = w_ref.shape[1]

    n_half = ybuf_ref.shape[2] // 2

    def _data_rdma(chunk, h):
        return pltpu.make_async_remote_copy(
            src_ref=ybuf_ref.at[chunk, :, pl.ds(h * n_half, n_half)],
            dst_ref=yex_ref.at[pl.ds(me * m_per, m_per), pl.ds(h * n_half, n_half)],
            send_sem=dsend.at[chunk * 2 + h],
            recv_sem=drecv.at[chunk * 2 + h],
            device_id=(perm_ref[chunk],),
            device_id_type=pl.DeviceIdType.MESH,
        )

    @pl.when(jnp.logical_and(j == 0, ki == 0))
    def _():
        barrier_sem = pltpu.get_barrier_semaphore()
        for d in range(1, N_DEV):
            peer = lax.rem(me + d, N_DEV)
            pl.semaphore_signal(
                barrier_sem, inc=1,
                device_id=(peer,), device_id_type=pl.DeviceIdType.MESH,
            )
        pl.semaphore_wait(barrier_sem, N_DEV - 1)
        amax_smem[0] = 0.0

    y = jnp.dot(
        x_ref[...].astype(jnp.bfloat16),
        w_ref[...].astype(jnp.bfloat16),
        preferred_element_type=jnp.float32,
    )
    amax_smem[0] = jnp.maximum(amax_smem[0], jnp.max(jnp.abs(y)))
    ybuf_ref[j, :, pl.ds(ki * bn, bn)] = y.astype(jnp.bfloat16)

    @pl.when(jnp.logical_and(ki == KI // 2 - 1, j < N_DEV - 1))
    def _():
        _data_rdma(j, 0).start()

    @pl.when(jnp.logical_and(ki == KI - 1, j < N_DEV - 1))
    def _():
        _data_rdma(j, 1).start()

    @pl.when(jnp.logical_and(ki == KI - 1, j == N_DEV - 1))
    def _():
        cp = pltpu.make_async_copy(
            ybuf_ref.at[j],
            yex_ref.at[pl.ds(me * m_per, m_per), :],
            copy_sem,
        )
        cp.start()

        amax_tile[...] = jnp.full_like(amax_tile, amax_smem[0])
        a_rdmas = []
        for d in range(1, N_DEV):
            peer = lax.rem(me + d, N_DEV)
            ar = pltpu.make_async_remote_copy(
                src_ref=amax_tile,
                dst_ref=amax_all.at[d],
                send_sem=asend.at[d],
                recv_sem=arecv.at[d],
                device_id=(peer,),
                device_id_type=pl.DeviceIdType.MESH,
            )
            ar.start()
            a_rdmas.append(ar)

        cp.wait()
        for c in range(N_DEV - 1):
            for h in range(2):
                _data_rdma(c, h).wait_send()
        for c in range(N_DEV - 1):
            for h in range(2):
                _data_rdma(c, h).wait_recv()
        for ar in a_rdmas:
            ar.wait()

        g = amax_smem[0]
        for d in range(1, N_DEV):
            g = jnp.maximum(g, amax_all[d, 0, 0])
        gmax_ref[...] = jnp.full_like(gmax_ref, g)


def _fused_gemm_a2a(perm, x, w):
    m_per, k = x.shape
    _, n = w.shape
    n_per = n // N_DEV
    bn = n_per // KI
    grid_spec = pltpu.PrefetchScalarGridSpec(
        num_scalar_prefetch=1,
        grid=(N_DEV, KI),
        in_specs=[
            pl.BlockSpec((m_per, k), lambda j, ki, perm: (0, 0)),
            pl.BlockSpec((k, bn), lambda j, ki, perm: (0, perm[j] * KI + ki)),
        ],
        out_specs=[
            pl.BlockSpec(memory_space=pl.ANY),
            pl.BlockSpec((8, 128), lambda j, ki, perm: (0, 0)),
        ],
        scratch_shapes=[
            pltpu.VMEM((N_DEV, m_per, n_per), jnp.bfloat16),
            pltpu.SMEM((1,), jnp.float32),
            pltpu.VMEM((8, 128), jnp.float32),
            pltpu.VMEM((N_DEV, 8, 128), jnp.float32),
            pltpu.SemaphoreType.DMA((2 * (N_DEV - 1),)),
            pltpu.SemaphoreType.DMA((2 * (N_DEV - 1),)),
            pltpu.SemaphoreType.DMA((N_DEV,)),
            pltpu.SemaphoreType.DMA((N_DEV,)),
            pltpu.SemaphoreType.DMA,
        ],
    )
    return pl.pallas_call(
        _fused_body,
        grid_spec=grid_spec,
        out_shape=[
            jax.ShapeDtypeStruct((N_DEV * m_per, n_per), jnp.bfloat16),
            jax.ShapeDtypeStruct((8, 128), jnp.float32),
        ],
        compiler_params=pltpu.CompilerParams(
            collective_id=0,
            vmem_limit_bytes=60 * 1024 * 1024,
        ),
    )(perm, x, w)


def _quant_body(y_ref, gmax_ref, out_ref):
    g = gmax_ref[0, 0]
    scale = g / 127.0
    y = y_ref[...].astype(jnp.float32)
    q = jnp.clip(jnp.round(y * (127.0 / g)), -127.0, 127.0)
    out_ref[...] = q * scale


def _quant(y_ex, gmax):
    m, n_per = y_ex.shape
    nb = 8
    bm = m // nb
    return pl.pallas_call(
        _quant_body,
        grid=(nb,),
        in_specs=[
            pl.BlockSpec((bm, n_per), lambda j: (j, 0)),
            pl.BlockSpec((8, 128), lambda j: (0, 0)),
        ],
        out_specs=pl.BlockSpec((bm, n_per), lambda j: (j, 0)),
        out_shape=jax.ShapeDtypeStruct((m, n_per), jnp.float32),
    )(y_ex, gmax)


def kernel(x, w_mat):
    me = lax.axis_index("i")
    perm = lax.rem(me + jnp.array([2, 1, 3, 0], dtype=jnp.int32), N_DEV)
    y_ex, gmax = _fused_gemm_a2a(perm, x, w_mat)
    return _quant(y_ex, gmax)
